# baseline (speedup 1.0000x reference)
import functools

import jax
import jax.numpy as jnp
from jax import lax
from jax.experimental import pallas as pl
from jax.experimental.pallas import tpu as pltpu
from jax.experimental.pallas import tpu_sc as plsc

NC = 2
NS = 16
LANES = 16
B = 128
SUPER = 6
BM = 1024


def _cdiv(a, b):
    return (a + b - 1) // b


def _agg_body(n_pad, w, nsup, src_hbm, dst_hbm, table_hbm, out_hbm,
              sbuf0, sbuf1, dbuf0, dbuf1, rows0, rows1, acc,
              gsem0, gsem1, ssem0, ssem1, sisem0, sisem1, disem0, disem1,
              zsem):
    c = lax.axis_index("c")
    s = lax.axis_index("s")
    tid = c * NS + s
    slab = n_pad // NS
    base_row = tid * (nsup * SUPER)

    sb, db = (sbuf0, sbuf1), (dbuf0, dbuf1)
    rw = (rows0, rows1)
    gs, ss = (gsem0, gsem1), (ssem0, ssem1)
    si, di = (sisem0, sisem1), (disem0, disem1)

    def fire_idx(hbm, t, buf, sem):
        pltpu.async_copy(hbm.at[pl.ds(base_row + t * SUPER, SUPER)], buf, sem)

    def drain_idx(hbm, buf, sem):
        pltpu.make_async_copy(hbm.at[pl.ds(base_row, SUPER)], buf, sem).wait()

    def fire_gathers(p):
        for j in range(SUPER):
            pltpu.async_copy(table_hbm.at[sb[p].at[j]], rw[p].at[j], gs[p])

    def drain_gathers(p):
        for j in range(SUPER):
            pltpu.make_async_copy(
                table_hbm.at[sb[p].at[j]], rw[p].at[j], gs[p]).wait()

    def fire_scatters(p):
        for j in range(SUPER):
            pltpu.async_copy(rw[p].at[j], acc.at[db[p].at[j]], ss[p], add=True)

    def drain_scatters(p):
        for j in range(SUPER):
            pltpu.make_async_copy(
                rw[p].at[j], acc.at[db[p].at[j]], ss[p]).wait()

    fire_idx(src_hbm, 0, sbuf0, sisem0)
    fire_idx(src_hbm, 1, sbuf1, sisem1)
    fire_idx(dst_hbm, 0, dbuf0, disem0)

    def zrow(i, carry):
        rows0[0, i, :] = jnp.zeros((LANES,), jnp.float32)
        return carry

    lax.fori_loop(0, B, zrow, 0)
    nz = slab // B

    def zchunk(k, carry):
        pltpu.async_copy(rows0.at[0],
                         acc.at[pl.ds(s * slab + k * B, B)], zsem)
        return carry

    def zdrain(k, carry):
        pltpu.make_async_copy(rows0.at[0],
                              acc.at[pl.ds(s * slab + k * B, B)], zsem).wait()
        return carry

    lax.fori_loop(0, nz, zchunk, 0)
    lax.fori_loop(0, nz, zdrain, 0)
    drain_idx(src_hbm, sbuf0, sisem0)
    fire_gathers(0)
    plsc.subcore_barrier()

    def sup(t, carry):
        def branch(p):
            q = 1 - p

            @pl.when(t >= 1)
            def _():
                drain_scatters(q)

            @pl.when(t + 1 < nsup)
            def _():
                fire_idx(dst_hbm, t + 1, db[q], di[q])
                drain_idx(src_hbm, sb[q], si[q])
                fire_gathers(q)

            drain_idx(dst_hbm, db[p], di[p])
            for j in range(SUPER):
                pltpu.make_async_copy(
                    table_hbm.at[sb[p].at[j]], rw[p].at[j], gs[p]).wait()
                pltpu.async_copy(
                    rw[p].at[j], acc.at[db[p].at[j]], ss[p], add=True)

            @pl.when(t + 2 < nsup)
            def _():
                fire_idx(src_hbm, t + 2, sb[p], si[p])

        @pl.when(t % 2 == 0)
        def _():
            branch(0)

        @pl.when(t % 2 == 1)
        def _():
            branch(1)

        return carry

    lax.fori_loop(0, nsup, sup, 0)
    p_last = (nsup - 1) % 2
    for j in range(SUPER):
        pltpu.make_async_copy(
            rw[p_last].at[j], acc.at[db[p_last].at[j]], ss[p_last]).wait()
    plsc.subcore_barrier()
    pltpu.sync_copy(acc.at[pl.ds(s * slab, slab)],
                    out_hbm.at[c, pl.ds(s * slab, slab)])


def _deg_body(n_pad, w, nsup, supb, dst_hbm, out_hbm,
              dbuf0, dbuf1, ones_v, zbuf, acc,
              ssem0, ssem1, disem0, disem1, zsem):
    c = lax.axis_index("c")
    s = lax.axis_index("s")
    tid = c * NS + s
    slab = n_pad // NS
    base_row = tid * (nsup * supb)

    db = (dbuf0, dbuf1)
    ss, di = (ssem0, ssem1), (disem0, disem1)

    def fire_idx(t, buf, sem):
        pltpu.async_copy(dst_hbm.at[pl.ds(base_row + t * supb, supb)],
                         buf, sem)

    def drain_idx(buf, sem):
        pltpu.make_async_copy(dst_hbm.at[pl.ds(base_row, supb)],
                              buf, sem).wait()

    def fire_scatters(p):
        for j in range(supb):
            pltpu.async_copy(ones_v, acc.at[db[p].at[j]], ss[p], add=True)

    def drain_scatters(p):
        for j in range(supb):
            pltpu.make_async_copy(
                ones_v, acc.at[db[p].at[j]], ss[p]).wait()

    fire_idx(0, dbuf0, disem0)

    def fill(i, carry):
        ones_v[i, :] = jnp.ones((LANES,), jnp.float32)
        zbuf[i, :] = jnp.zeros((LANES,), jnp.float32)
        return carry

    lax.fori_loop(0, B, fill, 0)
    nz = slab // B

    def zchunk(k, carry):
        pltpu.async_copy(zbuf, acc.at[pl.ds(s * slab + k * B, B)], zsem)
        return carry

    def zdrain(k, carry):
        pltpu.make_async_copy(zbuf,
                              acc.at[pl.ds(s * slab + k * B, B)], zsem).wait()
        return carry

    lax.fori_loop(0, nz, zchunk, 0)
    lax.fori_loop(0, nz, zdrain, 0)
    plsc.subcore_barrier()

    def sup(t, carry):
        def branch(p):
            q = 1 - p

            @pl.when(t >= 1)
            def _():
                drain_scatters(q)

            @pl.when(t + 1 < nsup)
            def _():
                fire_idx(t + 1, db[q], di[q])

            drain_idx(db[p], di[p])
            fire_scatters(p)

        @pl.when(t % 2 == 0)
        def _():
            branch(0)

        @pl.when(t % 2 == 1)
        def _():
            branch(1)

        return carry

    lax.fori_loop(0, nsup, sup, 0)
    p_last = (nsup - 1) % 2
    for j in range(supb):
        pltpu.make_async_copy(
            ones_v, acc.at[db[p_last].at[j]], ss[p_last]).wait()
    plsc.subcore_barrier()
    pltpu.sync_copy(acc.at[pl.ds(s * slab, slab)],
                    out_hbm.at[c, pl.ds(s * slab, slab)])


def _sc_mesh():
    return plsc.VectorSubcoreMesh(core_axis_name="c", subcore_axis_name="s",
                                  num_cores=NC, num_subcores=NS)


def _edge_aggregate(src2d, dst2d, table, n_pad, w):
    nsup = src2d.shape[0] // (NC * NS * SUPER)
    fn = pl.kernel(
        functools.partial(_agg_body, n_pad, w, nsup),
        out_type=jax.ShapeDtypeStruct((NC, n_pad, w), jnp.float32),
        mesh=_sc_mesh(),
        compiler_params=pltpu.CompilerParams(use_tc_tiling_on_sc=False),
        scratch_types=(
            [pltpu.VMEM((SUPER, B), jnp.int32)] * 4
            + [pltpu.VMEM((SUPER, B, w), jnp.float32)] * 2
            + [pltpu.VMEM_SHARED((n_pad, w), jnp.float32)]
            + [pltpu.SemaphoreType.DMA] * 9
        ),
    )
    return fn(src2d, dst2d, table)


SUPER_D = 24


def _degree_histogram(dst2d, n_pad, w):
    nsup = dst2d.shape[0] // (NC * NS * SUPER_D)
    fn = pl.kernel(
        functools.partial(_deg_body, n_pad, w, nsup, SUPER_D),
        out_type=jax.ShapeDtypeStruct((NC, n_pad, w), jnp.float32),
        mesh=_sc_mesh(),
        compiler_params=pltpu.CompilerParams(use_tc_tiling_on_sc=False),
        scratch_types=(
            [pltpu.VMEM((SUPER_D, B), jnp.int32)] * 2
            + [pltpu.VMEM((B, w), jnp.float32)] * 2
            + [pltpu.VMEM_SHARED((n_pad, w), jnp.float32)]
            + [pltpu.SemaphoreType.DMA] * 5
        ),
    )
    return fn(dst2d)


def _tc0_body(degp_ref, feat_ref, dis_ref, xs_ref):
    deg = degp_ref[0] + degp_ref[1] + 1.0
    dis = lax.rsqrt(deg)
    dis_ref[...] = dis
    xs_ref[...] = feat_ref[...] * dis


def _tc1_body(accp_ref, xs_ref, dis_ref, w1_ref, b1_ref, w2_ref, ys_ref):
    dis = dis_ref[...]
    agg = (accp_ref[0] + accp_ref[1] + xs_ref[...]) * dis
    h = jnp.dot(agg, w1_ref[...], preferred_element_type=jnp.float32)
    h = jnp.maximum(h + b1_ref[...], 0.0)
    ys = jnp.dot(h, w2_ref[...], preferred_element_type=jnp.float32)
    ys_ref[...] = ys * dis


def _tc2_body(accp_ref, ys_ref, dis_ref, b2_ref, out_ref):
    out_ref[...] = ((accp_ref[0] + accp_ref[1] + ys_ref[...]) * dis_ref[...]
                    + b2_ref[...])


def _pick_bmf(r_fold):
    for cand in range(min(512, r_fold), 7, -8):
        if r_fold % cand == 0:
            return cand
    return r_fold


def _row_spec(bmf):
    return pl.BlockSpec((bmf, 128), lambda i: (i, 0))


def _part_spec(bmf):
    return pl.BlockSpec((NC, bmf, 128), lambda i: (0, i, 0))


def _full_spec(shape):
    return pl.BlockSpec(shape, lambda i: tuple(0 for _ in shape))


def _tc0(degp_f, feat_f, r_fold):
    bmf = _pick_bmf(r_fold)
    grid = (r_fold // bmf,)
    return pl.pallas_call(
        _tc0_body,
        grid=grid,
        in_specs=[_part_spec(bmf), _row_spec(bmf)],
        out_specs=[_row_spec(bmf), _row_spec(bmf)],
        out_shape=[jax.ShapeDtypeStruct((r_fold, 128), jnp.float32),
                   jax.ShapeDtypeStruct((r_fold, 128), jnp.float32)],
    )(degp_f, feat_f)


def _tc1(accp_f, xs_f, dis_f, w1big, b1big, w2big, r_fold, d_hid_big):
    bmf = _pick_bmf(r_fold)
    grid = (r_fold // bmf,)
    return pl.pallas_call(
        _tc1_body,
        grid=grid,
        in_specs=[_part_spec(bmf), _row_spec(bmf), _row_spec(bmf),
                  _full_spec((128, d_hid_big)), _full_spec((1, d_hid_big)),
                  _full_spec((d_hid_big, 128))],
        out_specs=_row_spec(bmf),
        out_shape=jax.ShapeDtypeStruct((r_fold, 128), jnp.float32),
    )(accp_f, xs_f, dis_f, w1big, b1big, w2big)


def _tc2(accp_f, ys_f, dis_f, b2big, r_fold):
    bmf = _pick_bmf(r_fold)
    grid = (r_fold // bmf,)
    return pl.pallas_call(
        _tc2_body,
        grid=grid,
        in_specs=[_part_spec(bmf), _row_spec(bmf), _row_spec(bmf),
                  _full_spec((1, 128))],
        out_specs=_row_spec(bmf),
        out_shape=jax.ShapeDtypeStruct((r_fold, 128), jnp.float32),
    )(accp_f, ys_f, dis_f, b2big)


def kernel(feature, edge_index, edge_type, W1, b1, W2, b2):
    del edge_type
    n, d_in = feature.shape
    d_hid = W1.shape[1]
    d_out = W2.shape[1]
    e = edge_index.shape[1]
    w = d_in

    n_pad = _cdiv(n + 1, 2 * BM) * (2 * BM)
    e_gran = NC * NS * max(SUPER, SUPER_D) * B
    e_pad = _cdiv(e, e_gran) * e_gran

    src = edge_index[0].astype(jnp.int32)
    dst = edge_index[1].astype(jnp.int32)
    pad = e_pad - e
    if pad:
        pidx = jnp.arange(pad, dtype=jnp.int32)
        src = jnp.concatenate([src, pidx % n])
        dst = jnp.concatenate([dst, n + pidx % (n_pad - n)])
    src2d = src.reshape(-1, B)
    dst2d = dst.reshape(-1, B)

    f = 128 // w
    r_fold = n_pad // f
    d_hid_big = f * d_hid

    feat_f = jnp.concatenate(
        [feature, jnp.zeros((n_pad - n, d_in), jnp.float32)]
    ).reshape(r_fold, 128)
    eye_f = jnp.eye(f, dtype=jnp.float32)
    w1big = jnp.kron(eye_f, W1)
    b1big = jnp.tile(b1, f).reshape(1, d_hid_big)
    w2p = jnp.concatenate(
        [W2, jnp.zeros((d_hid, w - d_out), jnp.float32)], axis=1)
    w2big = jnp.kron(eye_f, w2p)
    b2p = jnp.concatenate([b2, jnp.zeros((w - d_out,), jnp.float32)])
    b2big = jnp.tile(b2p, f).reshape(1, 128)

    degp_f = _degree_histogram(dst2d, n_pad, w).reshape(NC, r_fold, 128)
    dis_f, xs_f = _tc0(degp_f, feat_f, r_fold)
    xs_sc = xs_f.reshape(n_pad, w)
    accb_f = _edge_aggregate(src2d, dst2d, xs_sc, n_pad, w).reshape(
        NC, r_fold, 128)
    ys_f = _tc1(accb_f, xs_f, dis_f, w1big, b1big, w2big, r_fold, d_hid_big)
    ys_sc = ys_f.reshape(n_pad, w)
    accc_f = _edge_aggregate(src2d, dst2d, ys_sc, n_pad, w).reshape(
        NC, r_fold, 128)
    out_f = _tc2(accc_f, ys_f, dis_f, b2big, r_fold)
    out3 = out_f.reshape(r_fold, f, w)[:, :, :d_out]
    return out3.reshape(n_pad, d_out)[:n]


# speedup vs baseline: 1.0004x; 1.0004x over previous
import functools

import jax
import jax.numpy as jnp
from jax import lax
from jax.experimental import pallas as pl
from jax.experimental.pallas import tpu as pltpu
from jax.experimental.pallas import tpu_sc as plsc

NC = 2
NS = 16
LANES = 16
B = 128
SUPER = 6
BM = 1024


def _cdiv(a, b):
    return (a + b - 1) // b


def _agg_body(n_pad, w, nsup, src_hbm, dst_hbm, table_hbm, out_hbm,
              sbuf0, sbuf1, dbuf0, dbuf1, rows0, rows1, acc,
              gsem0, gsem1, ssem0, ssem1, sisem0, sisem1, disem0, disem1,
              zsem):
    c = lax.axis_index("c")
    s = lax.axis_index("s")
    tid = c * NS + s
    slab = n_pad // NS
    base_row = tid * (nsup * SUPER)

    sb, db = (sbuf0, sbuf1), (dbuf0, dbuf1)
    rw = (rows0, rows1)
    gs, ss = (gsem0, gsem1), (ssem0, ssem1)
    si, di = (sisem0, sisem1), (disem0, disem1)

    def fire_idx(hbm, t, buf, sem):
        pltpu.async_copy(hbm.at[pl.ds(base_row + t * SUPER, SUPER)], buf, sem)

    def drain_idx(hbm, buf, sem):
        pltpu.make_async_copy(hbm.at[pl.ds(base_row, SUPER)], buf, sem).wait()

    def fire_gathers(p):
        for j in range(SUPER):
            pltpu.async_copy(table_hbm.at[sb[p].at[j]], rw[p].at[j], gs[p])

    def drain_gathers(p):
        for j in range(SUPER):
            pltpu.make_async_copy(
                table_hbm.at[sb[p].at[j]], rw[p].at[j], gs[p]).wait()

    def fire_scatters(p):
        for j in range(SUPER):
            pltpu.async_copy(rw[p].at[j], acc.at[db[p].at[j]], ss[p], add=True)

    def drain_scatters(p):
        for j in range(SUPER):
            pltpu.make_async_copy(
                rw[p].at[j], acc.at[db[p].at[j]], ss[p]).wait()

    fire_idx(src_hbm, 0, sbuf0, sisem0)
    fire_idx(src_hbm, 1, sbuf1, sisem1)
    fire_idx(dst_hbm, 0, dbuf0, disem0)

    def zrow(i, carry):
        rows0[0, i, :] = jnp.zeros((LANES,), jnp.float32)
        return carry

    lax.fori_loop(0, B, zrow, 0)
    nz = slab // B

    def zchunk(k, carry):
        pltpu.async_copy(rows0.at[0],
                         acc.at[pl.ds(s * slab + k * B, B)], zsem)
        return carry

    def zdrain(k, carry):
        pltpu.make_async_copy(rows0.at[0],
                              acc.at[pl.ds(s * slab + k * B, B)], zsem).wait()
        return carry

    lax.fori_loop(0, nz, zchunk, 0)
    lax.fori_loop(0, nz, zdrain, 0)
    drain_idx(src_hbm, sbuf0, sisem0)
    fire_gathers(0)
    plsc.subcore_barrier()

    def sup(t, carry):
        def branch(p):
            q = 1 - p

            @pl.when(t >= 1)
            def _():
                drain_scatters(q)

            @pl.when(t + 1 < nsup)
            def _():
                fire_idx(dst_hbm, t + 1, db[q], di[q])
                drain_idx(src_hbm, sb[q], si[q])
                fire_gathers(q)

            drain_idx(dst_hbm, db[p], di[p])
            for j in range(SUPER):
                pltpu.make_async_copy(
                    table_hbm.at[sb[p].at[j]], rw[p].at[j], gs[p]).wait()
                pltpu.async_copy(
                    rw[p].at[j], acc.at[db[p].at[j]], ss[p], add=True)

            @pl.when(t + 2 < nsup)
            def _():
                fire_idx(src_hbm, t + 2, sb[p], si[p])

        @pl.when(t % 2 == 0)
        def _():
            branch(0)

        @pl.when(t % 2 == 1)
        def _():
            branch(1)

        return carry

    lax.fori_loop(0, nsup, sup, 0)
    p_last = (nsup - 1) % 2
    for j in range(SUPER):
        pltpu.make_async_copy(
            rw[p_last].at[j], acc.at[db[p_last].at[j]], ss[p_last]).wait()
    plsc.subcore_barrier()
    pltpu.sync_copy(acc.at[pl.ds(s * slab, slab)],
                    out_hbm.at[c, pl.ds(s * slab, slab)])


def _deg_body(n_pad, w, nsup, supb, dst_hbm, out_hbm,
              dbuf0, dbuf1, ones_v, zbuf, acc,
              ssem0, ssem1, disem0, disem1, zsem):
    c = lax.axis_index("c")
    s = lax.axis_index("s")
    tid = c * NS + s
    slab = n_pad // NS
    base_row = tid * (nsup * supb)

    db = (dbuf0, dbuf1)
    ss, di = (ssem0, ssem1), (disem0, disem1)

    def fire_idx(t, buf, sem):
        pltpu.async_copy(dst_hbm.at[pl.ds(base_row + t * supb, supb)],
                         buf, sem)

    def drain_idx(buf, sem):
        pltpu.make_async_copy(dst_hbm.at[pl.ds(base_row, supb)],
                              buf, sem).wait()

    def fire_scatters(p):
        for j in range(supb):
            pltpu.async_copy(ones_v, acc.at[db[p].at[j]], ss[p], add=True)

    def drain_scatters(p):
        for j in range(supb):
            pltpu.make_async_copy(
                ones_v, acc.at[db[p].at[j]], ss[p]).wait()

    fire_idx(0, dbuf0, disem0)

    def fill(i, carry):
        ones_v[i, :] = jnp.ones((LANES,), jnp.float32)
        zbuf[i, :] = jnp.zeros((LANES,), jnp.float32)
        return carry

    lax.fori_loop(0, B, fill, 0)
    nz = slab // B

    def zchunk(k, carry):
        pltpu.async_copy(zbuf, acc.at[pl.ds(s * slab + k * B, B)], zsem)
        return carry

    def zdrain(k, carry):
        pltpu.make_async_copy(zbuf,
                              acc.at[pl.ds(s * slab + k * B, B)], zsem).wait()
        return carry

    lax.fori_loop(0, nz, zchunk, 0)
    lax.fori_loop(0, nz, zdrain, 0)
    plsc.subcore_barrier()

    def sup(t, carry):
        def branch(p):
            q = 1 - p

            @pl.when(t >= 1)
            def _():
                drain_scatters(q)

            @pl.when(t + 1 < nsup)
            def _():
                fire_idx(t + 1, db[q], di[q])

            drain_idx(db[p], di[p])
            fire_scatters(p)

        @pl.when(t % 2 == 0)
        def _():
            branch(0)

        @pl.when(t % 2 == 1)
        def _():
            branch(1)

        return carry

    lax.fori_loop(0, nsup, sup, 0)
    p_last = (nsup - 1) % 2
    for j in range(supb):
        pltpu.make_async_copy(
            ones_v, acc.at[db[p_last].at[j]], ss[p_last]).wait()
    plsc.subcore_barrier()
    pltpu.sync_copy(acc.at[pl.ds(s * slab, slab)],
                    out_hbm.at[c, pl.ds(s * slab, slab)])


def _sc_mesh():
    return plsc.VectorSubcoreMesh(core_axis_name="c", subcore_axis_name="s",
                                  num_cores=NC, num_subcores=NS)


def _edge_aggregate(src2d, dst2d, table, n_pad, w):
    nsup = src2d.shape[0] // (NC * NS * SUPER)
    fn = pl.kernel(
        functools.partial(_agg_body, n_pad, w, nsup),
        out_type=jax.ShapeDtypeStruct((NC, n_pad, w), jnp.float32),
        mesh=_sc_mesh(),
        compiler_params=pltpu.CompilerParams(use_tc_tiling_on_sc=False),
        scratch_types=(
            [pltpu.VMEM((SUPER, B), jnp.int32)] * 4
            + [pltpu.VMEM((SUPER, B, w), jnp.float32)] * 2
            + [pltpu.VMEM_SHARED((n_pad, w), jnp.float32)]
            + [pltpu.SemaphoreType.DMA] * 9
        ),
    )
    return fn(src2d, dst2d, table)


SUPER_D = 24


def _degree_histogram(dst2d, n_pad, w):
    nsup = dst2d.shape[0] // (NC * NS * SUPER_D)
    fn = pl.kernel(
        functools.partial(_deg_body, n_pad, w, nsup, SUPER_D),
        out_type=jax.ShapeDtypeStruct((NC, n_pad, w), jnp.float32),
        mesh=_sc_mesh(),
        compiler_params=pltpu.CompilerParams(use_tc_tiling_on_sc=False),
        scratch_types=(
            [pltpu.VMEM((SUPER_D, B), jnp.int32)] * 2
            + [pltpu.VMEM((B, w), jnp.float32)] * 2
            + [pltpu.VMEM_SHARED((n_pad, w), jnp.float32)]
            + [pltpu.SemaphoreType.DMA] * 5
        ),
    )
    return fn(dst2d)


def _tc0_body(degp_ref, feat_ref, dis_ref, xs_ref):
    deg = degp_ref[0] + degp_ref[1] + 1.0
    dis = lax.rsqrt(deg)
    dis_ref[...] = dis
    xs_ref[...] = feat_ref[...] * dis


def _tc1_body(accp_ref, xs_ref, dis_ref, w1_ref, b1_ref, w2_ref, ys_ref):
    dis = dis_ref[...]
    agg = (accp_ref[0] + accp_ref[1] + xs_ref[...]) * dis
    h = jnp.dot(agg, w1_ref[...], preferred_element_type=jnp.float32)
    h = jnp.maximum(h + b1_ref[...], 0.0)
    ys = jnp.dot(h, w2_ref[...], preferred_element_type=jnp.float32)
    ys_ref[...] = ys * dis


def _tc2_body(accp_ref, ys_ref, dis_ref, b2_ref, out_ref):
    out_ref[...] = ((accp_ref[0] + accp_ref[1] + ys_ref[...]) * dis_ref[...]
                    + b2_ref[...])


def _pick_bmf(r_fold):
    for cand in range(min(512, r_fold), 7, -8):
        if r_fold % cand == 0:
            return cand
    return r_fold


def _row_spec(bmf):
    return pl.BlockSpec((bmf, 128), lambda i: (i, 0))


def _part_spec(bmf):
    return pl.BlockSpec((NC, bmf, 128), lambda i: (0, i, 0))


def _full_spec(shape):
    return pl.BlockSpec(shape, lambda i: tuple(0 for _ in shape))


def _tc0(degp_f, feat_f, r_fold):
    bmf = _pick_bmf(r_fold)
    grid = (r_fold // bmf,)
    return pl.pallas_call(
        _tc0_body,
        grid=grid,
        in_specs=[_part_spec(bmf), _row_spec(bmf)],
        out_specs=[_row_spec(bmf), _row_spec(bmf)],
        out_shape=[jax.ShapeDtypeStruct((r_fold, 128), jnp.float32),
                   jax.ShapeDtypeStruct((r_fold, 128), jnp.float32)],
    )(degp_f, feat_f)


def _tc1(accp_f, xs_f, dis_f, w1big, b1big, w2big, r_fold, d_hid_big):
    bmf = _pick_bmf(r_fold)
    grid = (r_fold // bmf,)
    return pl.pallas_call(
        _tc1_body,
        grid=grid,
        in_specs=[_part_spec(bmf), _row_spec(bmf), _row_spec(bmf),
                  _full_spec((128, d_hid_big)), _full_spec((1, d_hid_big)),
                  _full_spec((d_hid_big, 128))],
        out_specs=_row_spec(bmf),
        out_shape=jax.ShapeDtypeStruct((r_fold, 128), jnp.float32),
    )(accp_f, xs_f, dis_f, w1big, b1big, w2big)


def _tc2(accp_f, ys_f, dis_f, b2big, r_fold):
    bmf = _pick_bmf(r_fold)
    grid = (r_fold // bmf,)
    return pl.pallas_call(
        _tc2_body,
        grid=grid,
        in_specs=[_part_spec(bmf), _row_spec(bmf), _row_spec(bmf),
                  _full_spec((1, 128))],
        out_specs=_row_spec(bmf),
        out_shape=jax.ShapeDtypeStruct((r_fold, 128), jnp.float32),
    )(accp_f, ys_f, dis_f, b2big)


def kernel(feature, edge_index, edge_type, W1, b1, W2, b2):
    del edge_type
    n, d_in = feature.shape
    d_hid = W1.shape[1]
    d_out = W2.shape[1]
    e = edge_index.shape[1]
    w = d_in

    n_pad = _cdiv(n + 1, BM) * BM
    e_gran = NC * NS * max(SUPER, SUPER_D) * B
    e_pad = _cdiv(e, e_gran) * e_gran

    src = edge_index[0].astype(jnp.int32)
    dst = edge_index[1].astype(jnp.int32)
    pad = e_pad - e
    if pad:
        pidx = jnp.arange(pad, dtype=jnp.int32)
        src = jnp.concatenate([src, pidx % n])
        dst = jnp.concatenate([dst, n + pidx % (n_pad - n)])
    src2d = src.reshape(-1, B)
    dst2d = dst.reshape(-1, B)

    f = 128 // w
    r_fold = n_pad // f
    d_hid_big = f * d_hid

    feat_f = jnp.concatenate(
        [feature, jnp.zeros((n_pad - n, d_in), jnp.float32)]
    ).reshape(r_fold, 128)
    eye_f = jnp.eye(f, dtype=jnp.float32)
    w1big = jnp.kron(eye_f, W1)
    b1big = jnp.tile(b1, f).reshape(1, d_hid_big)
    w2p = jnp.concatenate(
        [W2, jnp.zeros((d_hid, w - d_out), jnp.float32)], axis=1)
    w2big = jnp.kron(eye_f, w2p)
    b2p = jnp.concatenate([b2, jnp.zeros((w - d_out,), jnp.float32)])
    b2big = jnp.tile(b2p, f).reshape(1, 128)

    degp_f = _degree_histogram(dst2d, n_pad, w).reshape(NC, r_fold, 128)
    dis_f, xs_f = _tc0(degp_f, feat_f, r_fold)
    xs_sc = xs_f.reshape(n_pad, w)
    accb_f = _edge_aggregate(src2d, dst2d, xs_sc, n_pad, w).reshape(
        NC, r_fold, 128)
    ys_f = _tc1(accb_f, xs_f, dis_f, w1big, b1big, w2big, r_fold, d_hid_big)
    ys_sc = ys_f.reshape(n_pad, w)
    accc_f = _edge_aggregate(src2d, dst2d, ys_sc, n_pad, w).reshape(
        NC, r_fold, 128)
    out_f = _tc2(accc_f, ys_f, dis_f, b2big, r_fold)
    out3 = out_f.reshape(r_fold, f, w)[:, :, :d_out]
    return out3.reshape(n_pad, d_out)[:n]
